# Initial kernel scaffold; baseline (speedup 1.0000x reference)
import jax
import jax.numpy as jnp
from jax.experimental import pallas as pl


def kernel(input, emb_table, W, b):
    raise NotImplementedError("write your pallas kernel here")


# speedup vs baseline: 1.2310x; 1.2310x over previous
import functools

import jax
import jax.numpy as jnp
from jax import lax
from jax.experimental import pallas as pl
from jax.experimental.pallas import tpu as pltpu
from jax.experimental.pallas import tpu_sc as plsc

B = 10
S = 2752
E = 300
V = 147158
NW = 32
SBLK = S // NW
HALF = SBLK // 2
NFULL = E // 16
TAIL = E - NFULL * 16
TAIL_OFF = E - 16


def _sc_partials(idx3, table, w4):
    mesh = plsc.VectorSubcoreMesh(
        core_axis_name="c", subcore_axis_name="s",
        num_cores=2, num_subcores=16)

    @functools.partial(
        pl.kernel,
        out_type=jax.ShapeDtypeStruct((NW, 2, B, 16), jnp.float32),
        mesh=mesh,
        scratch_types=[
            pltpu.VMEM((B * 128,), jnp.int32),
            pltpu.VMEM((2, SBLK, E), jnp.float32),
            pltpu.VMEM((2, HALF, E), jnp.float32),
            pltpu.VMEM((2, B, 16), jnp.float32),
            pltpu.SemaphoreType.DMA,
            pltpu.SemaphoreType.DMA,
        ],
    )
    def k(idx_hbm, tab_hbm, w_hbm, out_hbm,
          idx_v, w_v, rows, out_v, sem0, sem1):
        wid = lax.axis_index("s") * 2 + lax.axis_index("c")
        sems = (sem0, sem1)

        pltpu.sync_copy(w_hbm.at[0, wid], w_v.at[0])
        pltpu.sync_copy(w_hbm.at[1, wid], w_v.at[1])
        for b in range(B):
            pltpu.sync_copy(idx_hbm.at[b, wid],
                            idx_v.at[pl.ds(b * 128, 128)])

        def fire(b, h):
            par = h % 2

            def body(s, _):
                iv = idx_v[pl.ds(b * 128 + h * HALF + s, 16)]
                pltpu.async_copy(tab_hbm.at[iv[0]],
                                 rows.at[par, s], sems[par])
                return 0

            lax.fori_loop(0, HALF, body, 0)

        def drain(h):
            par = h % 2

            def body(s, _):
                pltpu.make_async_copy(tab_hbm.at[0], rows.at[par, 0],
                                      sems[par]).wait()
                return 0

            lax.fori_loop(0, HALF, body, 0)

        lane = lax.iota(jnp.int32, 16)
        tmask = jnp.where(lane >= (16 - TAIL), 1.0, 0.0).astype(jnp.float32)
        zero = jnp.zeros((16,), jnp.float32)

        fire(0, 0)
        for b in range(B):
            for h in range(2):
                par = h % 2
                nb, nh = (b, h + 1) if h == 0 else (b + 1, 0)
                if nb < B:
                    fire(nb, nh)
                drain(h)

                def body(s, carry, par=par, h=h):
                    a0, a1 = carry
                    w_s = h * HALF + s
                    for kk in range(NFULL):
                        r = rows[par, s, pl.ds(kk * 16, 16)]
                        a0 = a0 + r * w_v[0, w_s, pl.ds(kk * 16, 16)]
                        a1 = a1 + r * w_v[1, w_s, pl.ds(kk * 16, 16)]
                    r = rows[par, s, pl.ds(TAIL_OFF, 16)] * tmask
                    a0 = a0 + r * w_v[0, w_s, pl.ds(TAIL_OFF, 16)]
                    a1 = a1 + r * w_v[1, w_s, pl.ds(TAIL_OFF, 16)]
                    return a0, a1

                init = (zero, zero) if h == 0 else (acc0, acc1)
                acc0, acc1 = lax.fori_loop(0, HALF, body, init)

            out_v[0, b] = acc0
            out_v[1, b] = acc1

        pltpu.sync_copy(out_v, out_hbm.at[wid])

    return k(idx3, table, w4)


def _tc_finish(kernel_partials, bias2):
    def body(p_ref, b_ref, o_ref):
        s = jnp.sum(p_ref[...], axis=(0, 3)) + b_ref[...]
        mx = jnp.max(s, axis=1, keepdims=True)
        lse = mx + jnp.log(jnp.sum(jnp.exp(s - mx), axis=1, keepdims=True))
        o_ref[...] = s - lse

    return pl.pallas_call(
        body,
        out_shape=jax.ShapeDtypeStruct((2, B), jnp.float32),
    )(kernel_partials, bias2)


def kernel(input, emb_table, W, b):
    idx3 = jnp.pad(input.reshape(B, NW, SBLK).astype(jnp.int32),
                   ((0, 0), (0, 0), (0, 128 - SBLK)))
    w4 = W.reshape(2, S, E).reshape(2, NW, SBLK, E)
    partials = _sc_partials(idx3, emb_table, w4)
    logp = _tc_finish(partials, b.reshape(2, 1))
    return logp.T


# speedup vs baseline: 1.2641x; 1.0269x over previous
import functools

import jax
import jax.numpy as jnp
from jax import lax
from jax.experimental import pallas as pl
from jax.experimental.pallas import tpu as pltpu
from jax.experimental.pallas import tpu_sc as plsc

B = 10
S = 2752
E = 300
V = 147158
NW = 32
SBLK = S // NW
QB = 8
NQ = 11
LASTQ = SBLK - QB * (NQ - 1)
NFULL = E // 16
TAIL = E - NFULL * 16
TAIL_OFF = E - 16


def _sc_partials(idx3, table, w4):
    mesh = plsc.VectorSubcoreMesh(
        core_axis_name="c", subcore_axis_name="s",
        num_cores=2, num_subcores=16)

    @functools.partial(
        pl.kernel,
        out_type=jax.ShapeDtypeStruct((NW, 2, B, 16), jnp.float32),
        mesh=mesh,
        scratch_types=[
            pltpu.VMEM((B * 128,), jnp.int32),
            pltpu.VMEM((2, 2, QB, E), jnp.float32),
            pltpu.VMEM((2, B, QB, E), jnp.float32),
            pltpu.VMEM((2, B, 16), jnp.float32),
            pltpu.SemaphoreType.DMA,
            pltpu.SemaphoreType.DMA,
            pltpu.SemaphoreType.DMA,
            pltpu.SemaphoreType.DMA,
        ],
    )
    def k(idx_hbm, tab_hbm, w_hbm, out_hbm,
          idx_v, w_v, rows, acc_v, rsem0, rsem1, wsem0, wsem1):
        wid = lax.axis_index("s") * 2 + lax.axis_index("c")
        rsems = (rsem0, rsem1)
        wsems = (wsem0, wsem1)

        for b in range(B):
            pltpu.sync_copy(idx_hbm.at[b, wid],
                            idx_v.at[pl.ds(b * 128, 128)])

        zero = jnp.zeros((16,), jnp.float32)
        for b in range(B):
            acc_v[0, b] = zero
            acc_v[1, b] = zero

        lane = lax.iota(jnp.int32, 16)
        tmask = jnp.where(lane >= (16 - TAIL), 1.0, 0.0).astype(jnp.float32)

        def fire_w(qbase, par, n):
            for c in range(2):
                pltpu.async_copy(w_hbm.at[c, wid, pl.ds(qbase, n)],
                                 w_v.at[par, c, pl.ds(0, n)], wsems[par])

        def drain_w(par, n):
            for c in range(2):
                pltpu.make_async_copy(
                    w_hbm.at[c, wid, pl.ds(0, n)],
                    w_v.at[par, c, pl.ds(0, n)], wsems[par]).wait()

        def fire_rows(qbase, par, n):
            def body(s, _):
                for b in range(B):
                    iv = idx_v[pl.ds(b * 128 + qbase + s, 16)]
                    pltpu.async_copy(tab_hbm.at[iv[0]],
                                     rows.at[par, b, s], rsems[par])
                return 0

            lax.fori_loop(0, n, body, 0)

        def drain_rows(par, n):
            def body(s, _):
                for b in range(B):
                    pltpu.make_async_copy(tab_hbm.at[0], rows.at[par, 0, 0],
                                          rsems[par]).wait()
                return 0

            lax.fori_loop(0, n, body, 0)

        def zero_w_tail(par):
            for c in range(2):
                for s in range(LASTQ, QB):
                    for kk in range(NFULL):
                        w_v[par, c, s, pl.ds(kk * 16, 16)] = zero
                    w_v[par, c, s, pl.ds(TAIL_OFF, 16)] = zero

        def compute(par):
            def body(s, _):
                w0 = [w_v[par, 0, s, pl.ds(kk * 16, 16)]
                      for kk in range(NFULL)]
                w0t = w_v[par, 0, s, pl.ds(TAIL_OFF, 16)]
                w1 = [w_v[par, 1, s, pl.ds(kk * 16, 16)]
                      for kk in range(NFULL)]
                w1t = w_v[par, 1, s, pl.ds(TAIL_OFF, 16)]
                for b in range(B):
                    a0 = acc_v[0, b]
                    a1 = acc_v[1, b]
                    for kk in range(NFULL):
                        r = rows[par, b, s, pl.ds(kk * 16, 16)]
                        a0 = a0 + r * w0[kk]
                        a1 = a1 + r * w1[kk]
                    r = rows[par, b, s, pl.ds(TAIL_OFF, 16)] * tmask
                    a0 = a0 + r * w0t
                    a1 = a1 + r * w1t
                    acc_v[0, b] = a0
                    acc_v[1, b] = a1
                return 0

            lax.fori_loop(0, QB, body, 0)

        fire_w(0, 0, QB)
        fire_rows(0, 0, QB)

        def pair(q2, _):
            qe = q2 * 2
            fire_w((qe + 1) * QB, 1, QB)
            fire_rows((qe + 1) * QB, 1, QB)
            drain_w(0, QB)
            drain_rows(0, QB)
            compute(0)
            @pl.when(q2 < 4)
            def _():
                fire_w((qe + 2) * QB, 0, QB)
                fire_rows((qe + 2) * QB, 0, QB)

            @pl.when(q2 == 4)
            def _():
                fire_w((NQ - 1) * QB, 0, LASTQ)
                fire_rows((NQ - 1) * QB, 0, LASTQ)

            drain_w(1, QB)
            drain_rows(1, QB)
            compute(1)
            return 0

        lax.fori_loop(0, 5, pair, 0)

        zero_w_tail(0)
        drain_w(0, LASTQ)
        drain_rows(0, LASTQ)
        compute(0)

        pltpu.sync_copy(acc_v, out_hbm.at[wid])

    return k(idx3, table, w4)


def _tc_finish(kernel_partials, bias2):
    def body(p_ref, b_ref, o_ref):
        s = jnp.sum(p_ref[...], axis=(0, 3)) + b_ref[...]
        mx = jnp.max(s, axis=1, keepdims=True)
        lse = mx + jnp.log(jnp.sum(jnp.exp(s - mx), axis=1, keepdims=True))
        o_ref[...] = s - lse

    return pl.pallas_call(
        body,
        out_shape=jax.ShapeDtypeStruct((2, B), jnp.float32),
    )(kernel_partials, bias2)


def kernel(input, emb_table, W, b):
    idx3 = jnp.pad(input.reshape(B, NW, SBLK).astype(jnp.int32),
                   ((0, 0), (0, 0), (0, 128 - SBLK)))
    w4 = W.reshape(2, S, E).reshape(2, NW, SBLK, E)
    partials = _sc_partials(idx3, emb_table, w4)
    logp = _tc_finish(partials, b.reshape(2, 1))
    return logp.T
